# Initial kernel scaffold; baseline (speedup 1.0000x reference)
import functools

import jax
import jax.numpy as jnp
from jax import lax
from jax.experimental import pallas as pl
from jax.experimental.pallas import tpu as pltpu
from jax.experimental.pallas import tpu_sc as plsc

NE = 320000
D = 128
NC, NS, L = 2, 16, 16
NW = NC * NS
E_PER_W = NE // NW
CHUNK = 400
NCHUNK = E_PER_W // CHUNK

_mesh = plsc.VectorSubcoreMesh(core_axis_name="c", subcore_axis_name="s")


@functools.partial(
    pl.kernel,
    mesh=_mesh,
    out_type=jax.ShapeDtypeStruct((NE,), jnp.float32),
    scratch_types=[
        pltpu.VMEM((CHUNK,), jnp.int32),
        pltpu.VMEM((CHUNK,), jnp.int32),
        pltpu.VMEM((CHUNK, D), jnp.float32),
        pltpu.VMEM((CHUNK, D), jnp.float32),
        pltpu.VMEM((CHUNK,), jnp.float32),
        pltpu.SemaphoreType.DMA,
        pltpu.SemaphoreType.DMA,
    ],
)
def _sc_dot_kernel(xp_hbm, xd_hbm, idx0_hbm, idx1_hbm, out_hbm,
                   idx0_v, idx1_v, rows0_v, rows1_v, out_v, sem0, sem1):
    wid = lax.axis_index("s") * NC + lax.axis_index("c")
    base_w = wid * E_PER_W

    def chunk_body(ci, carry):
        base = base_w + ci * CHUNK
        pltpu.sync_copy(idx0_hbm.at[pl.ds(base, CHUNK)], idx0_v)
        pltpu.sync_copy(idx1_hbm.at[pl.ds(base, CHUNK)], idx1_v)
        cp0 = pltpu.async_copy(xp_hbm.at[idx0_v], rows0_v, sem0)
        cp1 = pltpu.async_copy(xd_hbm.at[idx1_v], rows1_v, sem1)
        cp0.wait()
        cp1.wait()

        def edge_body(e, c):
            acc = rows0_v[e, pl.ds(0, L)] * rows1_v[e, pl.ds(0, L)]
            for j in range(1, D // L):
                a = rows0_v[e, pl.ds(j * L, L)]
                b = rows1_v[e, pl.ds(j * L, L)]
                acc = acc + a * b
            out_v[e] = jnp.sum(acc)
            return c

        lax.fori_loop(0, CHUNK, edge_body, 0)
        pltpu.sync_copy(out_v, out_hbm.at[pl.ds(base, CHUNK)])
        return carry

    lax.fori_loop(0, NCHUNK, chunk_body, 0)


def kernel(x_patient, x_drug, edge_label_index):
    return _sc_dot_kernel(x_patient, x_drug,
                          edge_label_index[0], edge_label_index[1])


# speedup vs baseline: 3.6093x; 3.6093x over previous
import functools

import jax
import jax.numpy as jnp
from jax import lax
from jax.experimental import pallas as pl
from jax.experimental.pallas import tpu as pltpu
from jax.experimental.pallas import tpu_sc as plsc

NE = 320000
D = 128
NC, NS, L = 2, 16, 16
NW = NC * NS
E_PER_W = NE // NW
CHUNK = 400
NCHUNK = E_PER_W // CHUNK

_mesh = plsc.VectorSubcoreMesh(core_axis_name="c", subcore_axis_name="s")


def _lane_take(x, idx):
    dnums = lax.GatherDimensionNumbers(
        offset_dims=(), collapsed_slice_dims=(0,), start_index_map=(0,))
    return lax.gather(x, idx[:, None], dnums, slice_sizes=(1,),
                      mode=lax.GatherScatterMode.PROMISE_IN_BOUNDS)


@functools.partial(
    pl.kernel,
    mesh=_mesh,
    out_type=jax.ShapeDtypeStruct((NE,), jnp.float32),
    scratch_types=[
        pltpu.VMEM((CHUNK,), jnp.int32),
        pltpu.VMEM((CHUNK,), jnp.int32),
        pltpu.VMEM((CHUNK, D), jnp.float32),
        pltpu.VMEM((CHUNK, D), jnp.float32),
        pltpu.VMEM((CHUNK,), jnp.float32),
        pltpu.SemaphoreType.DMA,
        pltpu.SemaphoreType.DMA,
    ],
)
def _sc_dot_kernel(xp_hbm, xd_hbm, idx0_hbm, idx1_hbm, out_hbm,
                   idx0_v, idx1_v, rows0_v, rows1_v, out_v, sem0, sem1):
    wid = lax.axis_index("s") * NC + lax.axis_index("c")
    base_w = wid * E_PER_W

    def chunk_body(ci, carry):
        base = base_w + ci * CHUNK
        pltpu.sync_copy(idx0_hbm.at[pl.ds(base, CHUNK)], idx0_v)
        pltpu.sync_copy(idx1_hbm.at[pl.ds(base, CHUNK)], idx1_v)
        cp0 = pltpu.async_copy(xp_hbm.at[idx0_v], rows0_v, sem0)
        cp1 = pltpu.async_copy(xd_hbm.at[idx1_v], rows1_v, sem1)
        cp0.wait()
        cp1.wait()

        lanes = lax.iota(jnp.int32, L)
        perms = [lanes ^ (L >> (p + 1)) for p in range(4)]

        def group_body(g, c):
            res = jnp.zeros((L,), jnp.float32)
            for k in range(L):
                e = g * L + k
                acc = rows0_v[e, pl.ds(0, L)] * rows1_v[e, pl.ds(0, L)]
                for j in range(1, D // L):
                    a = rows0_v[e, pl.ds(j * L, L)]
                    b = rows1_v[e, pl.ds(j * L, L)]
                    acc = acc + a * b
                for p in perms:
                    acc = acc + _lane_take(acc, p)
                res = jnp.where(lanes == k, acc, res)
            out_v[pl.ds(g * L, L)] = res
            return c

        lax.fori_loop(0, CHUNK // L, group_body, 0)
        pltpu.sync_copy(out_v, out_hbm.at[pl.ds(base, CHUNK)])
        return carry

    lax.fori_loop(0, NCHUNK, chunk_body, 0)


def kernel(x_patient, x_drug, edge_label_index):
    return _sc_dot_kernel(x_patient, x_drug,
                          edge_label_index[0], edge_label_index[1])


# speedup vs baseline: 7.9630x; 2.2062x over previous
import functools

import jax
import jax.numpy as jnp
from jax import lax
from jax.experimental import pallas as pl
from jax.experimental.pallas import tpu as pltpu
from jax.experimental.pallas import tpu_sc as plsc

NE = 320000
D = 128
W = D // 2
NC, NS, L = 2, 16, 16
NW = NC * NS
E_PER_W = NE // NW
CHUNK = 400
NCHUNK = E_PER_W // CHUNK

_mesh = plsc.VectorSubcoreMesh(core_axis_name="c", subcore_axis_name="s")


def _lane_take(x, idx):
    dnums = lax.GatherDimensionNumbers(
        offset_dims=(), collapsed_slice_dims=(0,), start_index_map=(0,))
    return lax.gather(x, idx[:, None], dnums, slice_sizes=(1,),
                      mode=lax.GatherScatterMode.PROMISE_IN_BOUNDS)


@functools.partial(
    pl.kernel,
    mesh=_mesh,
    out_type=jax.ShapeDtypeStruct((NE,), jnp.float32),
    compiler_params=pltpu.CompilerParams(needs_layout_passes=False,
                                         use_tc_tiling_on_sc=False),
    scratch_types=[
        pltpu.VMEM((CHUNK,), jnp.int32),
        pltpu.VMEM((CHUNK,), jnp.int32),
        pltpu.VMEM((CHUNK,), jnp.int32),
        pltpu.VMEM((CHUNK,), jnp.int32),
        pltpu.VMEM((CHUNK, W), jnp.float32),
        pltpu.VMEM((CHUNK, W), jnp.float32),
        pltpu.VMEM((CHUNK, W), jnp.float32),
        pltpu.VMEM((CHUNK, W), jnp.float32),
        pltpu.VMEM((CHUNK,), jnp.float32),
        pltpu.SemaphoreType.DMA,
        pltpu.SemaphoreType.DMA,
        pltpu.SemaphoreType.DMA,
        pltpu.SemaphoreType.DMA,
    ],
)
def _sc_dot_kernel(xp_hbm, xd_hbm, idx0_hbm, idx1_hbm, out_hbm,
                   i0a, i1a, i0b, i1b, r0a, r1a, r0b, r1b, out_v,
                   spa, sda, spb, sdb):
    wid = lax.axis_index("s") * NC + lax.axis_index("c")
    base_w = wid * E_PER_W

    lanes = lax.iota(jnp.int32, L)
    perms = [lanes ^ (L >> (p + 1)) for p in range(4)]

    def issue(ci, i0, i1, r0, r1, sp, sd):
        base = base_w + ci * CHUNK
        pltpu.sync_copy(idx0_hbm.at[pl.ds(base, CHUNK)], i0)
        pltpu.sync_copy(idx1_hbm.at[pl.ds(base, CHUNK)], i1)
        pltpu.async_copy(xp_hbm.at[i0], r0, sp)
        pltpu.async_copy(xd_hbm.at[i1], r1, sd)

    def consume(ci, i0, i1, r0, r1, sp, sd):
        pltpu.make_async_copy(xp_hbm.at[i0], r0, sp).wait()
        pltpu.make_async_copy(xd_hbm.at[i1], r1, sd).wait()

        def group_body(g, c):
            res = jnp.zeros((L,), jnp.float32)
            for k in range(L):
                e = g * L + k
                p = []
                for j in range(4):
                    a = plsc.bitcast(r0[e, pl.ds(j * L, L)], jnp.bfloat16)
                    b = plsc.bitcast(r1[e, pl.ds(j * L, L)], jnp.bfloat16)
                    p.append(a * b)
                s = (p[0] + p[1]) + (p[2] + p[3])
                lo, hi = plsc.unpack(s, format=plsc.PackFormat.INTERLEAVED)
                acc = lo + hi
                for pm in perms:
                    acc = acc + _lane_take(acc, pm)
                res = jnp.where(lanes == k, acc, res)
            out_v[pl.ds(g * L, L)] = res
            return c

        lax.fori_loop(0, CHUNK // L, group_body, 0)
        base = base_w + ci * CHUNK
        pltpu.sync_copy(out_v, out_hbm.at[pl.ds(base, CHUNK)])

    bufa = (i0a, i1a, r0a, r1a, spa, sda)
    bufb = (i0b, i1b, r0b, r1b, spb, sdb)

    issue(0, *bufa)

    def pair_body(g, carry):
        ci = 2 * g
        issue(ci + 1, *bufb)
        consume(ci, *bufa)

        @pl.when(ci + 2 < NCHUNK)
        def _():
            issue(ci + 2, *bufa)

        consume(ci + 1, *bufb)
        return carry

    lax.fori_loop(0, NCHUNK // 2, pair_body, 0)
    consume(NCHUNK - 1, *bufa)


def kernel(x_patient, x_drug, edge_label_index):
    xp = lax.bitcast_convert_type(
        x_patient.astype(jnp.bfloat16).reshape(-1, W, 2), jnp.float32)
    xd = lax.bitcast_convert_type(
        x_drug.astype(jnp.bfloat16).reshape(-1, W, 2), jnp.float32)
    return _sc_dot_kernel(xp, xd, edge_label_index[0], edge_label_index[1])


# speedup vs baseline: 9.5400x; 1.1980x over previous
import functools

import jax
import jax.numpy as jnp
from jax import lax
from jax.experimental import pallas as pl
from jax.experimental.pallas import tpu as pltpu
from jax.experimental.pallas import tpu_sc as plsc

NE = 320000
D = 128
W = D // 2
NC, NS, L = 2, 16, 16
NW = NC * NS
E_PER_W = NE // NW
CHUNK = 400
NCHUNK = E_PER_W // CHUNK

_mesh = plsc.VectorSubcoreMesh(core_axis_name="c", subcore_axis_name="s")


def _lane_take(x, idx):
    dnums = lax.GatherDimensionNumbers(
        offset_dims=(), collapsed_slice_dims=(0,), start_index_map=(0,))
    return lax.gather(x, idx[:, None], dnums, slice_sizes=(1,),
                      mode=lax.GatherScatterMode.PROMISE_IN_BOUNDS)


@functools.partial(
    pl.kernel,
    mesh=_mesh,
    out_type=jax.ShapeDtypeStruct((NE,), jnp.float32),
    compiler_params=pltpu.CompilerParams(needs_layout_passes=False,
                                         use_tc_tiling_on_sc=False),
    scratch_types=[
        pltpu.VMEM((E_PER_W,), jnp.int32),
        pltpu.VMEM((E_PER_W,), jnp.int32),
        pltpu.VMEM((CHUNK, W), jnp.float32),
        pltpu.VMEM((CHUNK, W), jnp.float32),
        pltpu.VMEM((CHUNK, W), jnp.float32),
        pltpu.VMEM((CHUNK, W), jnp.float32),
        pltpu.VMEM((CHUNK,), jnp.float32),
        pltpu.SemaphoreType.DMA,
        pltpu.SemaphoreType.DMA,
        pltpu.SemaphoreType.DMA,
        pltpu.SemaphoreType.DMA,
    ],
)
def _sc_dot_kernel(xp_hbm, xd_hbm, idx0_hbm, idx1_hbm, out_hbm,
                   i0w, i1w, r0a, r1a, r0b, r1b, out_v,
                   spa, sda, spb, sdb):
    wid = lax.axis_index("s") * NC + lax.axis_index("c")
    base_w = wid * E_PER_W
    pltpu.sync_copy(idx0_hbm.at[pl.ds(base_w, E_PER_W)], i0w)
    pltpu.sync_copy(idx1_hbm.at[pl.ds(base_w, E_PER_W)], i1w)

    lanes = lax.iota(jnp.int32, L)
    perms = [lanes ^ (L >> (p + 1)) for p in range(4)]

    def issue(ci, r0, r1, sp, sd):
        off = ci * CHUNK
        pltpu.async_copy(xp_hbm.at[i0w.at[pl.ds(off, CHUNK)]], r0, sp)
        pltpu.async_copy(xd_hbm.at[i1w.at[pl.ds(off, CHUNK)]], r1, sd)

    def consume(ci, r0, r1, sp, sd):
        off = ci * CHUNK
        pltpu.make_async_copy(xp_hbm.at[i0w.at[pl.ds(off, CHUNK)]], r0,
                              sp).wait()
        pltpu.make_async_copy(xd_hbm.at[i1w.at[pl.ds(off, CHUNK)]], r1,
                              sd).wait()

        def group_body(g, c):
            res = jnp.zeros((L,), jnp.float32)
            for k in range(L):
                e = g * L + k
                p = []
                for j in range(4):
                    a = plsc.bitcast(r0[e, pl.ds(j * L, L)], jnp.bfloat16)
                    b = plsc.bitcast(r1[e, pl.ds(j * L, L)], jnp.bfloat16)
                    p.append(a * b)
                s = (p[0] + p[1]) + (p[2] + p[3])
                lo, hi = plsc.unpack(s, format=plsc.PackFormat.INTERLEAVED)
                acc = lo + hi
                for pm in perms:
                    acc = acc + _lane_take(acc, pm)
                res = jnp.where(lanes == k, acc, res)
            out_v[pl.ds(g * L, L)] = res
            return c

        lax.fori_loop(0, CHUNK // L, group_body, 0)
        base = base_w + ci * CHUNK
        pltpu.sync_copy(out_v, out_hbm.at[pl.ds(base, CHUNK)])

    bufa = (r0a, r1a, spa, sda)
    bufb = (r0b, r1b, spb, sdb)

    issue(0, *bufa)

    def pair_body(g, carry):
        ci = 2 * g
        issue(ci + 1, *bufb)
        consume(ci, *bufa)

        @pl.when(ci + 2 < NCHUNK)
        def _():
            issue(ci + 2, *bufa)

        consume(ci + 1, *bufb)
        return carry

    lax.fori_loop(0, NCHUNK // 2, pair_body, 0)
    consume(NCHUNK - 1, *bufa)


def kernel(x_patient, x_drug, edge_label_index):
    xp = lax.bitcast_convert_type(
        x_patient.astype(jnp.bfloat16).reshape(-1, W, 2), jnp.float32)
    xd = lax.bitcast_convert_type(
        x_drug.astype(jnp.bfloat16).reshape(-1, W, 2), jnp.float32)
    return _sc_dot_kernel(xp, xd, edge_label_index[0], edge_label_index[1])


# speedup vs baseline: 14.7616x; 1.5473x over previous
import functools

import jax
import jax.numpy as jnp
from jax import lax
from jax.experimental import pallas as pl
from jax.experimental.pallas import tpu as pltpu
from jax.experimental.pallas import tpu_sc as plsc

NE = 320000
D = 128
W = D // 2
NC, NS, L = 2, 16, 16
NW = NC * NS
E_PER_W = NE // NW
CHUNK = 400
NCHUNK = E_PER_W // CHUNK

_mesh = plsc.VectorSubcoreMesh(core_axis_name="c", subcore_axis_name="s")


def _lane_take(x, idx):
    dnums = lax.GatherDimensionNumbers(
        offset_dims=(), collapsed_slice_dims=(0,), start_index_map=(0,))
    return lax.gather(x, idx[:, None], dnums, slice_sizes=(1,),
                      mode=lax.GatherScatterMode.PROMISE_IN_BOUNDS)


@functools.partial(
    pl.kernel,
    mesh=_mesh,
    out_type=jax.ShapeDtypeStruct((NE,), jnp.float32),
    compiler_params=pltpu.CompilerParams(needs_layout_passes=False,
                                         use_tc_tiling_on_sc=False),
    scratch_types=[
        pltpu.VMEM((E_PER_W,), jnp.int32),
        pltpu.VMEM((E_PER_W,), jnp.int32),
        pltpu.VMEM((CHUNK, W), jnp.float32),
        pltpu.VMEM((CHUNK, W), jnp.float32),
        pltpu.VMEM((CHUNK, W), jnp.float32),
        pltpu.VMEM((CHUNK, W), jnp.float32),
        pltpu.VMEM((CHUNK,), jnp.float32),
        pltpu.SemaphoreType.DMA,
        pltpu.SemaphoreType.DMA,
        pltpu.SemaphoreType.DMA,
        pltpu.SemaphoreType.DMA,
    ],
)
def _sc_dot_kernel(xp_hbm, xd_hbm, idx_hbm, out_hbm,
                   i0w, i1w, r0a, r1a, r0b, r1b, out_v,
                   spa, sda, spb, sdb):
    wid = lax.axis_index("s") * NC + lax.axis_index("c")
    base_w = wid * E_PER_W
    pltpu.sync_copy(idx_hbm.at[0, pl.ds(base_w, E_PER_W)], i0w)
    pltpu.sync_copy(idx_hbm.at[1, pl.ds(base_w, E_PER_W)], i1w)

    lanes = lax.iota(jnp.int32, L)
    perms = [lanes ^ (L >> (p + 1)) for p in range(4)]

    def issue(ci, r0, r1, sp, sd):
        off = ci * CHUNK
        pltpu.async_copy(xp_hbm.at[i0w.at[pl.ds(off, CHUNK)]], r0, sp)
        pltpu.async_copy(xd_hbm.at[i1w.at[pl.ds(off, CHUNK)]], r1, sd)

    def consume(ci, r0, r1, sp, sd):
        off = ci * CHUNK
        pltpu.make_async_copy(xp_hbm.at[i0w.at[pl.ds(off, CHUNK)]], r0,
                              sp).wait()
        pltpu.make_async_copy(xd_hbm.at[i1w.at[pl.ds(off, CHUNK)]], r1,
                              sd).wait()

        def group_body(g, c):
            res = jnp.zeros((L,), jnp.float32)
            for k in range(L):
                e = g * L + k
                p = []
                for j in range(4):
                    a = plsc.bitcast(r0[e, pl.ds(j * L, L)], jnp.bfloat16)
                    b = plsc.bitcast(r1[e, pl.ds(j * L, L)], jnp.bfloat16)
                    p.append(a * b)
                s = (p[0] + p[1]) + (p[2] + p[3])
                lo, hi = plsc.unpack(s, format=plsc.PackFormat.INTERLEAVED)
                acc = lo + hi
                for pm in perms:
                    acc = acc + _lane_take(acc, pm)
                res = jnp.where(lanes == k, acc, res)
            out_v[pl.ds(g * L, L)] = res
            return c

        lax.fori_loop(0, CHUNK // L, group_body, 0)
        base = base_w + ci * CHUNK
        pltpu.sync_copy(out_v, out_hbm.at[pl.ds(base, CHUNK)])

    bufa = (r0a, r1a, spa, sda)
    bufb = (r0b, r1b, spb, sdb)

    issue(0, *bufa)

    def pair_body(g, carry):
        ci = 2 * g
        issue(ci + 1, *bufb)
        consume(ci, *bufa)

        @pl.when(ci + 2 < NCHUNK)
        def _():
            issue(ci + 2, *bufa)

        consume(ci + 1, *bufb)
        return carry

    lax.fori_loop(0, NCHUNK // 2, pair_body, 0)
    consume(NCHUNK - 1, *bufa)


def _pack_bf16(x):
    u = lax.bitcast_convert_type(x.astype(jnp.bfloat16), jnp.uint16)
    u = u.astype(jnp.uint32)
    return lax.bitcast_convert_type(u[:, :W] | (u[:, W:] << 16), jnp.float32)


def kernel(x_patient, x_drug, edge_label_index):
    return _sc_dot_kernel(_pack_bf16(x_patient), _pack_bf16(x_drug),
                          edge_label_index)


# speedup vs baseline: 15.0334x; 1.0184x over previous
import functools

import jax
import jax.numpy as jnp
from jax import lax
from jax.experimental import pallas as pl
from jax.experimental.pallas import tpu as pltpu
from jax.experimental.pallas import tpu_sc as plsc

NE = 320000
D = 128
W = D // 2
NC, NS, L = 2, 16, 16
NW = NC * NS
E_PER_W = NE // NW
CHUNK = 400
NCHUNK = E_PER_W // CHUNK

_mesh = plsc.VectorSubcoreMesh(core_axis_name="c", subcore_axis_name="s")


def _lane_take(x, idx):
    dnums = lax.GatherDimensionNumbers(
        offset_dims=(), collapsed_slice_dims=(0,), start_index_map=(0,))
    return lax.gather(x, idx[:, None], dnums, slice_sizes=(1,),
                      mode=lax.GatherScatterMode.PROMISE_IN_BOUNDS)


@functools.partial(
    pl.kernel,
    mesh=_mesh,
    out_type=jax.ShapeDtypeStruct((NE,), jnp.float32),
    compiler_params=pltpu.CompilerParams(needs_layout_passes=False,
                                         use_tc_tiling_on_sc=False),
    scratch_types=[
        pltpu.VMEM((E_PER_W,), jnp.int32),
        pltpu.VMEM((E_PER_W,), jnp.int32),
        pltpu.VMEM((CHUNK, W), jnp.float32),
        pltpu.VMEM((CHUNK, W), jnp.float32),
        pltpu.VMEM((CHUNK, W), jnp.float32),
        pltpu.VMEM((CHUNK, W), jnp.float32),
        pltpu.VMEM((CHUNK,), jnp.float32),
        pltpu.SemaphoreType.DMA,
        pltpu.SemaphoreType.DMA,
        pltpu.SemaphoreType.DMA,
        pltpu.SemaphoreType.DMA,
    ],
)
def _sc_dot_kernel(xp_hbm, xd_hbm, idx_hbm, out_hbm,
                   i0w, i1w, r0a, r1a, r0b, r1b, out_v,
                   spa, sda, spb, sdb):
    wid = lax.axis_index("s") * NC + lax.axis_index("c")
    base_w = wid * E_PER_W
    pltpu.sync_copy(idx_hbm.at[0, pl.ds(base_w, E_PER_W)], i0w)
    pltpu.sync_copy(idx_hbm.at[1, pl.ds(base_w, E_PER_W)], i1w)

    lanes = lax.iota(jnp.int32, L)
    masks = [((lanes >> b) & 1) == 0 for b in range(4)]
    perms = [lanes ^ (1 << b) for b in range(4)]

    def issue(ci, r0, r1, sp, sd):
        off = ci * CHUNK
        pltpu.async_copy(xp_hbm.at[i0w.at[pl.ds(off, CHUNK)]], r0, sp)
        pltpu.async_copy(xd_hbm.at[i1w.at[pl.ds(off, CHUNK)]], r1, sd)

    def consume(ci, r0, r1, sp, sd):
        off = ci * CHUNK
        pltpu.make_async_copy(xp_hbm.at[i0w.at[pl.ds(off, CHUNK)]], r0,
                              sp).wait()
        pltpu.make_async_copy(xd_hbm.at[i1w.at[pl.ds(off, CHUNK)]], r1,
                              sd).wait()

        def group_body(g, c):
            accs = []
            for k in range(L):
                e = g * L + k
                p = []
                for j in range(4):
                    a = plsc.bitcast(r0[e, pl.ds(j * L, L)], jnp.bfloat16)
                    b = plsc.bitcast(r1[e, pl.ds(j * L, L)], jnp.bfloat16)
                    p.append(a * b)
                s = (p[0] + p[1]) + (p[2] + p[3])
                lo, hi = plsc.unpack(s, format=plsc.PackFormat.INTERLEAVED)
                accs.append(lo + hi)
            for b in range(4):
                m, pm = masks[b], perms[b]
                accs = [jnp.where(m, u, v) + _lane_take(jnp.where(m, v, u), pm)
                        for u, v in zip(accs[0::2], accs[1::2])]
            out_v[pl.ds(g * L, L)] = accs[0]
            return c

        lax.fori_loop(0, CHUNK // L, group_body, 0)
        base = base_w + ci * CHUNK
        pltpu.sync_copy(out_v, out_hbm.at[pl.ds(base, CHUNK)])

    bufa = (r0a, r1a, spa, sda)
    bufb = (r0b, r1b, spb, sdb)

    issue(0, *bufa)

    def pair_body(g, carry):
        ci = 2 * g
        issue(ci + 1, *bufb)
        consume(ci, *bufa)

        @pl.when(ci + 2 < NCHUNK)
        def _():
            issue(ci + 2, *bufa)

        consume(ci + 1, *bufb)
        return carry

    lax.fori_loop(0, NCHUNK // 2, pair_body, 0)
    consume(NCHUNK - 1, *bufa)


def _pack_bf16(x):
    u = lax.bitcast_convert_type(x.astype(jnp.bfloat16), jnp.uint16)
    u = u.astype(jnp.uint32)
    return lax.bitcast_convert_type(u[:, :W] | (u[:, W:] << 16), jnp.float32)


def kernel(x_patient, x_drug, edge_label_index):
    return _sc_dot_kernel(_pack_bf16(x_patient), _pack_bf16(x_drug),
                          edge_label_index)
